# baseline (speedup 1.0000x reference)
import jax
import jax.numpy as jnp
from jax.experimental import pallas as pl
from jax.experimental.pallas import tpu as pltpu

_NUM_SAMPLE = 2048
_G = 128
_C = 128


def _fps_kernel(pts_ref, outx_ref, outy_ref, outz_ref, dists_ref):
    _, NBLK, B, C = pts_ref.shape
    S = outx_ref.shape[0] * _G

    lane = jax.lax.broadcasted_iota(jnp.int32, (B, _G), 1)

    dists_ref[...] = jnp.full((NBLK, B, C), 1e10, dtype=jnp.float32)

    fx0 = pts_ref[0, 0, :, C - 1:C]
    fy0 = pts_ref[1, 0, :, C - 1:C]
    fz0 = pts_ref[2, 0, :, C - 1:C]
    buf0 = jnp.zeros((B, _G), dtype=jnp.float32)

    neg = jnp.full((B, C), -1e30, dtype=jnp.float32)
    zero = jnp.zeros((B, C), dtype=jnp.float32)

    def body(i, carry):
        fx, fy, fz, bx, by, bz = carry
        col = jax.lax.rem(i, _G)
        g = jax.lax.div(i, _G)
        hit = lane == col
        bx = jnp.where(hit, fx, bx)
        by = jnp.where(hit, fy, by)
        bz = jnp.where(hit, fz, bz)
        outx_ref[pl.ds(g, 1)] = bx.reshape(1, B, _G)
        outy_ref[pl.ds(g, 1)] = by.reshape(1, B, _G)
        outz_ref[pl.ds(g, 1)] = bz.reshape(1, B, _G)

        def chunk(j, acc):
            amax, ax, ay, az = acc
            xc = pts_ref[0, j]
            yc = pts_ref[1, j]
            zc = pts_ref[2, j]
            dx = xc - fx
            dy = yc - fy
            dz = zc - fz
            d = (dx * dx + dz * dz) + dy * dy
            nd = jnp.minimum(dists_ref[j], d)
            dists_ref[j] = nd
            cmp = nd > amax
            amax = jnp.where(cmp, nd, amax)
            ax = jnp.where(cmp, xc, ax)
            ay = jnp.where(cmp, yc, ay)
            az = jnp.where(cmp, zc, az)
            return (amax, ax, ay, az)

        amax, ax, ay, az = jax.lax.fori_loop(
            0, NBLK, chunk, (neg, zero, zero, zero), unroll=8)

        lstar = jnp.argmax(amax, axis=1).astype(jnp.int32)[:, None]
        oh = lane == lstar
        nfx = jnp.max(jnp.where(oh, ax, -1e30), axis=1, keepdims=True)
        nfy = jnp.max(jnp.where(oh, ay, -1e30), axis=1, keepdims=True)
        nfz = jnp.max(jnp.where(oh, az, -1e30), axis=1, keepdims=True)
        return (nfx, nfy, nfz, bx, by, bz)

    jax.lax.fori_loop(0, S, body, (fx0, fy0, fz0, buf0, buf0, buf0),
                      unroll=False)


def _run(points):
    B, N, _ = points.shape
    S = _NUM_SAMPLE
    nblk = N // _C
    pts = points.transpose(2, 0, 1).reshape(3, B, _C, nblk)
    pts = pts.transpose(0, 3, 1, 2)[:, :, :, ::-1]

    plane = jax.ShapeDtypeStruct((S // _G, B, _G), jnp.float32)
    return pl.pallas_call(
        _fps_kernel,
        out_shape=(plane, plane, plane),
        scratch_shapes=[pltpu.VMEM((nblk, B, _C), jnp.float32)],
    )(pts)


def kernel(points):
    B, _, _ = points.shape
    S = _NUM_SAMPLE
    ox, oy, oz = _run(points)
    samples = jnp.stack([ox, oy, oz], axis=-1)
    return samples.transpose(1, 0, 2, 3).reshape(B, S, 3)


# speedup vs baseline: 1.1752x; 1.1752x over previous
import jax
import jax.numpy as jnp
from jax.experimental import pallas as pl
from jax.experimental.pallas import tpu as pltpu

_NUM_SAMPLE = 2048
_G = 128
_C = 128
_H = 8


def _fps_kernel(pts_ref, outx_ref, outy_ref, outz_ref, dists_ref):
    _, NBLK, B, C = pts_ref.shape
    S = outx_ref.shape[0] * _G

    lane = jax.lax.broadcasted_iota(jnp.int32, (_H, _G), 1)

    dists_ref[...] = jnp.full((NBLK, B, C), 1e10, dtype=jnp.float32)

    neg = jnp.full((_H, C), -1e30, dtype=jnp.float32)
    zero = jnp.zeros((_H, C), dtype=jnp.float32)
    buf0 = jnp.zeros((_H, _G), dtype=jnp.float32)

    INL = 72

    def sweep(lo, fx, fy, fz):
        def chunk(j, acc):
            amax, ax, ay, az = acc
            xc = pts_ref[0, j, lo:lo + _H, :]
            yc = pts_ref[1, j, lo:lo + _H, :]
            zc = pts_ref[2, j, lo:lo + _H, :]
            dx = xc - fx
            dy = yc - fy
            dz = zc - fz
            d = (dx * dx + dz * dz) + dy * dy
            nd = jnp.minimum(dists_ref[j, lo:lo + _H, :], d)
            dists_ref[j, lo:lo + _H, :] = nd
            cmp = nd > amax
            amax = jnp.where(cmp, nd, amax)
            ax = jnp.where(cmp, xc, ax)
            ay = jnp.where(cmp, yc, ay)
            az = jnp.where(cmp, zc, az)
            return (amax, ax, ay, az)

        acc = (neg, zero, zero, zero)
        for j in range(INL):
            acc = chunk(j, acc)
        return jax.lax.fori_loop(INL, NBLK, chunk, acc, unroll=8)

    def finale(acc):
        amax, ax, ay, az = acc
        lstar = jnp.argmax(amax, axis=1).astype(jnp.int32)[:, None]
        oh = lane == lstar
        fx = jnp.max(jnp.where(oh, ax, -1e30), axis=1, keepdims=True)
        fy = jnp.max(jnp.where(oh, ay, -1e30), axis=1, keepdims=True)
        fz = jnp.max(jnp.where(oh, az, -1e30), axis=1, keepdims=True)
        return fx, fy, fz

    fxA0 = pts_ref[0, 0, 0:_H, C - 1:C]
    fyA0 = pts_ref[1, 0, 0:_H, C - 1:C]
    fzA0 = pts_ref[2, 0, 0:_H, C - 1:C]
    amaxB0 = jnp.where(lane == C - 1, 1.0, 0.0).astype(jnp.float32)
    accB0 = (amaxB0, pts_ref[0, 0, _H:B, :], pts_ref[1, 0, _H:B, :],
             pts_ref[2, 0, _H:B, :])

    def body(i, carry):
        fxA, fyA, fzA, amaxB, axB, ayB, azB, bufs = carry
        bxA, byA, bzA, bxB, byB, bzB = bufs
        col = jax.lax.rem(i, _G)
        g = jax.lax.div(i, _G)
        hit = lane == col

        fxB, fyB, fzB = finale((amaxB, axB, ayB, azB))

        bxA = jnp.where(hit, fxA, bxA)
        byA = jnp.where(hit, fyA, byA)
        bzA = jnp.where(hit, fzA, bzA)
        outx_ref[pl.ds(g, 1), 0:_H] = bxA.reshape(1, _H, _G)
        outy_ref[pl.ds(g, 1), 0:_H] = byA.reshape(1, _H, _G)
        outz_ref[pl.ds(g, 1), 0:_H] = bzA.reshape(1, _H, _G)
        accA = sweep(0, fxA, fyA, fzA)

        nfxA, nfyA, nfzA = finale(accA)

        bxB = jnp.where(hit, fxB, bxB)
        byB = jnp.where(hit, fyB, byB)
        bzB = jnp.where(hit, fzB, bzB)
        outx_ref[pl.ds(g, 1), _H:B] = bxB.reshape(1, _H, _G)
        outy_ref[pl.ds(g, 1), _H:B] = byB.reshape(1, _H, _G)
        outz_ref[pl.ds(g, 1), _H:B] = bzB.reshape(1, _H, _G)
        namaxB, naxB, nayB, nazB = sweep(_H, fxB, fyB, fzB)

        return (nfxA, nfyA, nfzA, namaxB, naxB, nayB, nazB,
                (bxA, byA, bzA, bxB, byB, bzB))

    jax.lax.fori_loop(
        0, S, body,
        (fxA0, fyA0, fzA0) + accB0 + ((buf0,) * 6,),
        unroll=False)


def _run(points):
    B, N, _ = points.shape
    S = _NUM_SAMPLE
    nblk = N // _C
    pts = points.transpose(2, 0, 1).reshape(3, B, _C, nblk)
    pts = pts.transpose(0, 3, 1, 2)[:, :, :, ::-1]

    plane = jax.ShapeDtypeStruct((S // _G, B, _G), jnp.float32)
    return pl.pallas_call(
        _fps_kernel,
        out_shape=(plane, plane, plane),
        scratch_shapes=[pltpu.VMEM((nblk, B, _C), jnp.float32)],
    )(pts)


def kernel(points):
    B, _, _ = points.shape
    S = _NUM_SAMPLE
    ox, oy, oz = _run(points)
    samples = jnp.stack([ox, oy, oz], axis=-1)
    return samples.transpose(1, 0, 2, 3).reshape(B, S, 3)


# speedup vs baseline: 1.2123x; 1.0316x over previous
import jax
import jax.numpy as jnp
from jax.experimental import pallas as pl
from jax.experimental.pallas import tpu as pltpu

_NUM_SAMPLE = 2048
_G = 128
_C = 128
_H = 8


def _fps_kernel(pts_ref, outx_ref, outy_ref, outz_ref, dists_ref):
    _, NBLK, B, C = pts_ref.shape
    S = outx_ref.shape[0] * _G

    lane = jax.lax.broadcasted_iota(jnp.int32, (_H, _G), 1)

    dists_ref[...] = jnp.full((NBLK, B, C), 1e10, dtype=jnp.float32)

    neg = jnp.full((_H, C), -1e30, dtype=jnp.float32)
    zero = jnp.zeros((_H, C), dtype=jnp.float32)
    buf0 = jnp.zeros((_H, _G), dtype=jnp.float32)

    INL = 128

    def sweep(lo, fx, fy, fz):
        def chunk(j, acc):
            amax, ax, ay, az = acc
            xc = pts_ref[0, j, lo:lo + _H, :]
            yc = pts_ref[1, j, lo:lo + _H, :]
            zc = pts_ref[2, j, lo:lo + _H, :]
            dx = xc - fx
            dy = yc - fy
            dz = zc - fz
            d = (dx * dx + dz * dz) + dy * dy
            nd = jnp.minimum(dists_ref[j, lo:lo + _H, :], d)
            dists_ref[j, lo:lo + _H, :] = nd
            cmp = nd > amax
            amax = jnp.where(cmp, nd, amax)
            ax = jnp.where(cmp, xc, ax)
            ay = jnp.where(cmp, yc, ay)
            az = jnp.where(cmp, zc, az)
            return (amax, ax, ay, az)

        acc = (neg, zero, zero, zero)
        for j in range(INL):
            acc = chunk(j, acc)
        if INL == NBLK:
            return acc
        return jax.lax.fori_loop(INL, NBLK, chunk, acc, unroll=8)

    def finale(acc):
        amax, ax, ay, az = acc
        lstar = jnp.argmax(amax, axis=1).astype(jnp.int32)[:, None]
        oh = lane == lstar
        fx = jnp.max(jnp.where(oh, ax, -1e30), axis=1, keepdims=True)
        fy = jnp.max(jnp.where(oh, ay, -1e30), axis=1, keepdims=True)
        fz = jnp.max(jnp.where(oh, az, -1e30), axis=1, keepdims=True)
        return fx, fy, fz

    fxA0 = pts_ref[0, 0, 0:_H, C - 1:C]
    fyA0 = pts_ref[1, 0, 0:_H, C - 1:C]
    fzA0 = pts_ref[2, 0, 0:_H, C - 1:C]
    amaxB0 = jnp.where(lane == C - 1, 1.0, 0.0).astype(jnp.float32)
    accB0 = (amaxB0, pts_ref[0, 0, _H:B, :], pts_ref[1, 0, _H:B, :],
             pts_ref[2, 0, _H:B, :])

    def body(i, carry):
        fxA, fyA, fzA, amaxB, axB, ayB, azB, bufs = carry
        bxA, byA, bzA, bxB, byB, bzB = bufs
        col = jax.lax.rem(i, _G)
        g = jax.lax.div(i, _G)
        hit = lane == col

        fxB, fyB, fzB = finale((amaxB, axB, ayB, azB))

        bxA = jnp.where(hit, fxA, bxA)
        byA = jnp.where(hit, fyA, byA)
        bzA = jnp.where(hit, fzA, bzA)
        outx_ref[pl.ds(g, 1), 0:_H] = bxA.reshape(1, _H, _G)
        outy_ref[pl.ds(g, 1), 0:_H] = byA.reshape(1, _H, _G)
        outz_ref[pl.ds(g, 1), 0:_H] = bzA.reshape(1, _H, _G)
        accA = sweep(0, fxA, fyA, fzA)

        nfxA, nfyA, nfzA = finale(accA)

        bxB = jnp.where(hit, fxB, bxB)
        byB = jnp.where(hit, fyB, byB)
        bzB = jnp.where(hit, fzB, bzB)
        outx_ref[pl.ds(g, 1), _H:B] = bxB.reshape(1, _H, _G)
        outy_ref[pl.ds(g, 1), _H:B] = byB.reshape(1, _H, _G)
        outz_ref[pl.ds(g, 1), _H:B] = bzB.reshape(1, _H, _G)
        namaxB, naxB, nayB, nazB = sweep(_H, fxB, fyB, fzB)

        return (nfxA, nfyA, nfzA, namaxB, naxB, nayB, nazB,
                (bxA, byA, bzA, bxB, byB, bzB))

    jax.lax.fori_loop(
        0, S, body,
        (fxA0, fyA0, fzA0) + accB0 + ((buf0,) * 6,),
        unroll=False)


def _run(points):
    B, N, _ = points.shape
    S = _NUM_SAMPLE
    nblk = N // _C
    pts = points.transpose(2, 0, 1).reshape(3, B, _C, nblk)
    pts = pts.transpose(0, 3, 1, 2)[:, :, :, ::-1]

    plane = jax.ShapeDtypeStruct((S // _G, B, _G), jnp.float32)
    return pl.pallas_call(
        _fps_kernel,
        out_shape=(plane, plane, plane),
        scratch_shapes=[pltpu.VMEM((nblk, B, _C), jnp.float32)],
    )(pts)


def kernel(points):
    B, _, _ = points.shape
    S = _NUM_SAMPLE
    ox, oy, oz = _run(points)
    samples = jnp.stack([ox, oy, oz], axis=-1)
    return samples.transpose(1, 0, 2, 3).reshape(B, S, 3)
